# baseline (speedup 1.0000x reference)
import jax
import jax.numpy as jnp
from jax import lax
from jax.experimental import pallas as pl
from jax.experimental.pallas import tpu as pltpu

_B, _H, _W, _C = 4, 512, 512, 32
_PAD = 3


def _focus_body(img_ref, out_ref, tmp_ref):
    for _img_i in range(8):
        _focus_one_image(img_ref, out_ref, tmp_ref, _img_i)


def _focus_one_image(img_ref, out_ref, tmp_ref, n):
    f32, i32 = jnp.float32, jnp.int32
    H, W = _H, _W
    img = img_ref[n]

    col_max = jnp.max(img, axis=0, keepdims=True)
    row_max = jnp.max(img, axis=1, keepdims=True)
    ax = jnp.round(jax.nn.sigmoid(col_max)).astype(i32)
    ay = jnp.round(jax.nn.sigmoid(row_max)).astype(i32)
    wxv = lax.broadcasted_iota(i32, (1, W), 1)
    wyv = lax.broadcasted_iota(i32, (H, 1), 0)
    m1 = jnp.max(ax * (W - 1 - wxv), axis=1, keepdims=True)
    xm = jnp.where(m1 > 0, W - 1 - m1, 0)
    xM = jnp.max(ax * wxv, axis=1, keepdims=True)
    m3 = jnp.max(ay * (H - 1 - wyv), axis=0, keepdims=True)
    ym = jnp.where(m3 > 0, H - 1 - m3, 0)
    yM = jnp.max(ay * wyv, axis=0, keepdims=True)

    hc = jnp.maximum(yM - ym, 1)
    wc = jnp.maximum(xM - xm, 1)
    hcf = hc.astype(f32)
    wcf = wc.astype(f32)
    zh = jnp.maximum(H - 2 * _PAD, hc).astype(f32)
    zw = jnp.maximum(W - 2 * _PAD, wc).astype(f32)
    scale = jnp.minimum(zh / hcf, zw / wcf)
    rh = jnp.round(scale * hcf).astype(i32)
    rw = jnp.round(scale * wcf).astype(i32)

    pad_t = jnp.maximum(0, (H - rh) // 2)
    crop_t = jnp.maximum(0, (rh - H) // 2)
    pad_l = jnp.maximum(0, (W - rw) // 2)
    crop_l = jnp.maximum(0, (rw - W) // 2)

    oi = lax.broadcasted_iota(i32, (H, 1), 0)
    ri = oi - pad_t + crop_t
    valid_r = (ri >= 0) & (ri < rh)
    oj = lax.broadcasted_iota(i32, (1, W), 1)
    rj = oj - pad_l + crop_l
    valid_c = (rj >= 0) & (rj < rw)

    ident = ((rh == hc) & (rw == wc)
             & (ym - pad_t + crop_t == 0) & (xm - pad_l + crop_l == 0))
    ident_s = ident.astype(i32)[0, 0]

    @pl.when(ident_s == 1)
    def _fast():
        out_ref[n] = jnp.where(valid_r & valid_c, img, 0.0)

    @pl.when(ident_s == 0)
    def _general():
        ratio_y = hcf / rh.astype(f32)
        sy = jnp.minimum(ri.astype(f32) * ratio_y, hcf - 1.0)
        s_y = jnp.where(valid_r, ym.astype(f32) + sy, -2.0 * H)

        i0v = lax.broadcasted_iota(i32, (4, 1), 0) * 128
        ri0v = jnp.clip(i0v - pad_t + crop_t, 0, rh - 1)
        sy0v = jnp.minimum(ri0v.astype(f32) * ratio_y, hcf - 1.0)
        y00v = jnp.clip(jnp.floor(sy0v).astype(i32), 0, hc - 1)
        basev = (jnp.clip(ym + y00v, 0, H - 256) // 8) * 8
        basevf = basev.astype(f32)

        d_row = lax.broadcasted_iota(i32, (1, 256), 1).astype(f32)
        for t in range(4):
            i0 = 128 * t
            rb = pl.multiple_of(basev[t, 0], 8)
            s_yb = lax.slice(s_y, (i0, 0), (i0 + 128, 1))
            Ay_t = jnp.maximum(
                1.0 - jnp.abs((s_yb - basevf[t, 0]) - d_row), 0.0)
            img_win = img_ref[n, pl.ds(rb, 256), :]
            tmp_ref[i0:i0 + 128, :] = jnp.dot(
                Ay_t, img_win, preferred_element_type=f32)

        sx = jnp.minimum(rj.astype(f32) * (wcf / rw.astype(f32)), wcf - 1.0)
        s_x = jnp.where(valid_c, xm.astype(f32) + sx, -2.0 * W)
        llv = lax.broadcasted_iota(i32, (W, 1), 0).astype(f32)
        AxT = jnp.maximum(1.0 - jnp.abs(s_x - llv), 0.0)
        out_ref[n] = jnp.dot(tmp_ref[...], AxT, preferred_element_type=f32)


def _focus_chunk(imgs):
    n = imgs.shape[0]
    return pl.pallas_call(
        _focus_body,
        grid=(n // 8,),
        in_specs=[pl.BlockSpec((8, _H, _W), lambda g: (g, 0, 0))],
        out_specs=pl.BlockSpec((8, _H, _W), lambda g: (g, 0, 0)),
        out_shape=jax.ShapeDtypeStruct((n, _H, _W), jnp.float32),
        scratch_shapes=[pltpu.VMEM((_H, _W), jnp.float32)],
        compiler_params=pltpu.CompilerParams(
            dimension_semantics=("arbitrary",),
            vmem_limit_bytes=48 * 1024 * 1024,
        ),
        name="focus2d",
    )(imgs)


def kernel(inputs):
    outs = []
    for lo in range(0, _B, 2):
        part = jnp.transpose(
            inputs[lo:lo + 2], (0, 3, 1, 2)).reshape(2 * _C, _H, _W)
        o = _focus_chunk(part)
        outs.append(jnp.transpose(
            o.reshape(2, _C, _H, _W), (0, 2, 3, 1)))
    return jnp.concatenate(outs, axis=0)


# speedup vs baseline: 1.6753x; 1.6753x over previous
import jax
import jax.numpy as jnp
from jax import lax
from jax.experimental import pallas as pl
from jax.experimental.pallas import tpu as pltpu

_B, _H, _W, _C = 4, 512, 512, 32
_PAD = 3


def _focus_body(img_ref, out_ref, tmp_ref):
    for _img_i in range(8):
        _focus_one_image(img_ref, out_ref, tmp_ref, _img_i)


def _focus_one_image(img_ref, out_ref, tmp_ref, n):
    f32, i32 = jnp.float32, jnp.int32
    H, W = _H, _W
    img = img_ref[n]

    col_max = jnp.max(img, axis=0, keepdims=True)
    row_max = jnp.max(img, axis=1, keepdims=True)
    ax = jnp.round(jax.nn.sigmoid(col_max)).astype(i32)
    ay = jnp.round(jax.nn.sigmoid(row_max)).astype(i32)
    wxv = lax.broadcasted_iota(i32, (1, W), 1)
    wyv = lax.broadcasted_iota(i32, (H, 1), 0)
    m1 = jnp.max(ax * (W - 1 - wxv), axis=1, keepdims=True)
    xm = jnp.where(m1 > 0, W - 1 - m1, 0)
    xM = jnp.max(ax * wxv, axis=1, keepdims=True)
    m3 = jnp.max(ay * (H - 1 - wyv), axis=0, keepdims=True)
    ym = jnp.where(m3 > 0, H - 1 - m3, 0)
    yM = jnp.max(ay * wyv, axis=0, keepdims=True)

    hc = jnp.maximum(yM - ym, 1)
    wc = jnp.maximum(xM - xm, 1)
    hcf = hc.astype(f32)
    wcf = wc.astype(f32)
    zh = jnp.maximum(H - 2 * _PAD, hc).astype(f32)
    zw = jnp.maximum(W - 2 * _PAD, wc).astype(f32)
    scale = jnp.minimum(zh / hcf, zw / wcf)
    rh = jnp.round(scale * hcf).astype(i32)
    rw = jnp.round(scale * wcf).astype(i32)

    pad_t = jnp.maximum(0, (H - rh) // 2)
    crop_t = jnp.maximum(0, (rh - H) // 2)
    pad_l = jnp.maximum(0, (W - rw) // 2)
    crop_l = jnp.maximum(0, (rw - W) // 2)

    oi = lax.broadcasted_iota(i32, (H, 1), 0)
    ri = oi - pad_t + crop_t
    valid_r = (ri >= 0) & (ri < rh)
    oj = lax.broadcasted_iota(i32, (1, W), 1)
    rj = oj - pad_l + crop_l
    valid_c = (rj >= 0) & (rj < rw)

    ident = ((rh == hc) & (rw == wc)
             & (ym - pad_t + crop_t == 0) & (xm - pad_l + crop_l == 0))
    ident_s = ident.astype(i32)[0, 0]

    @pl.when(ident_s == 1)
    def _fast():
        out_ref[n] = img * valid_r.astype(f32) * valid_c.astype(f32)

    @pl.when(ident_s == 0)
    def _general():
        ratio_y = hcf / rh.astype(f32)
        sy = jnp.minimum(ri.astype(f32) * ratio_y, hcf - 1.0)
        s_y = jnp.where(valid_r, ym.astype(f32) + sy, -2.0 * H)

        i0v = lax.broadcasted_iota(i32, (4, 1), 0) * 128
        ri0v = jnp.clip(i0v - pad_t + crop_t, 0, rh - 1)
        sy0v = jnp.minimum(ri0v.astype(f32) * ratio_y, hcf - 1.0)
        y00v = jnp.clip(jnp.floor(sy0v).astype(i32), 0, hc - 1)
        basev = (jnp.clip(ym + y00v, 0, H - 256) // 8) * 8
        basevf = basev.astype(f32)

        d_row = lax.broadcasted_iota(i32, (1, 256), 1).astype(f32)
        for t in range(4):
            i0 = 128 * t
            rb = pl.multiple_of(basev[t, 0], 8)
            s_yb = lax.slice(s_y, (i0, 0), (i0 + 128, 1))
            Ay_t = jnp.maximum(
                1.0 - jnp.abs((s_yb - basevf[t, 0]) - d_row), 0.0)
            img_win = img_ref[n, pl.ds(rb, 256), :]
            tmp_ref[i0:i0 + 128, :] = jnp.dot(
                Ay_t, img_win, preferred_element_type=f32)

        sx = jnp.minimum(rj.astype(f32) * (wcf / rw.astype(f32)), wcf - 1.0)
        s_x = jnp.where(valid_c, xm.astype(f32) + sx, -2.0 * W)
        llv = lax.broadcasted_iota(i32, (W, 1), 0).astype(f32)
        AxT = jnp.maximum(1.0 - jnp.abs(s_x - llv), 0.0)
        out_ref[n] = jnp.dot(tmp_ref[...], AxT, preferred_element_type=f32)


def kernel(inputs):
    imgs = jnp.transpose(inputs, (0, 3, 1, 2)).reshape(_B * _C, _H, _W)
    out = pl.pallas_call(
        _focus_body,
        grid=(_B * _C // 8,),
        in_specs=[pl.BlockSpec((8, _H, _W), lambda g: (g, 0, 0))],
        out_specs=pl.BlockSpec((8, _H, _W), lambda g: (g, 0, 0)),
        out_shape=jax.ShapeDtypeStruct((_B * _C, _H, _W), jnp.float32),
        scratch_shapes=[pltpu.VMEM((_H, _W), jnp.float32)],
        compiler_params=pltpu.CompilerParams(
            dimension_semantics=("arbitrary",),
            vmem_limit_bytes=48 * 1024 * 1024,
        ),
        name="focus2d",
    )(imgs)
    return jnp.transpose(out.reshape(_B, _C, _H, _W), (0, 2, 3, 1))
